# baseline (speedup 1.0000x reference)
import jax
import jax.numpy as jnp
from jax import lax
from jax.experimental import pallas as pl
from jax.experimental.pallas import tpu as pltpu
from jax.experimental.pallas import tpu_sc as plsc

_B, _N, _G, _M = 8, 8192, 512, 32
_L = 16
_QUART = _N // 4
_RPW = _G // 4
_CAP = 512
_NV = _CAP // _L


def _permute(v, idx):
    return lax.gather(
        v, idx[:, None],
        lax.GatherDimensionNumbers(offset_dims=(), collapsed_slice_dims=(0,),
                                   start_index_map=(0,)),
        (1,), mode=lax.GatherScatterMode.PROMISE_IN_BOUNDS)


def _cmpex(vk, vi, stride, lanes, dirm=None):
    pidx = lanes ^ stride
    pk = _permute(vk, pidx)
    pi = _permute(vi, pidx)
    less = (pk < vk) | ((pk == vk) & (pi < vi))
    is_high = (lanes & stride) != 0
    take_self = less == is_high
    if dirm is not None:
        take_self = take_self != dirm
    return jnp.where(take_self, vk, pk), jnp.where(take_self, vi, pi)


def _sort16(vk, vi, lanes):
    for k in (2, 4, 8, 16):
        dirm = (lanes & k) != 0
        stride = k // 2
        while stride >= 1:
            vk, vi = _cmpex(vk, vi, stride, lanes, dirm)
            stride //= 2
    return vk, vi


def _minmaxpair(ak, ai, bk, bi):
    bless = (bk < ak) | ((bk == ak) & (bi < ai))
    return (jnp.where(bless, bk, ak), jnp.where(bless, bi, ai),
            jnp.where(bless, ak, bk), jnp.where(bless, ai, bi))


def _merge(R, sk, si, lanes):
    R0k, R0i, R1k, R1i = R
    rev = 15 - lanes
    rk = _permute(sk, rev)
    ri = _permute(si, rev)
    l1k, l1i, _, _ = _minmaxpair(R1k, R1i, rk, ri)
    ak, ai, bk, bi = _minmaxpair(R0k, R0i, l1k, l1i)
    for stride in (8, 4, 2, 1):
        ak, ai = _cmpex(ak, ai, stride, lanes)
        bk, bi = _cmpex(bk, bi, stride, lanes)
    return ak, ai, bk, bi


def _select32(cd, cix, lanes, nv):
    def mbody(v, R):
        kd = cd[pl.ds(v * _L, _L)]
        ki = cix[pl.ds(v * _L, _L)]
        worst = R[2][15]

        def do(Rin):
            sk, si = _sort16(kd, ki, lanes)
            return _merge(Rin, sk, si, lanes)

        return lax.cond(jnp.min(kd) <= worst, do, lambda Rin: Rin, R)

    inf = jnp.full((_L,), jnp.inf, jnp.float32)
    zero = jnp.zeros((_L,), jnp.int32)
    return lax.fori_loop(0, nv, mbody, (inf, zero, inf, zero))


def _group_body(x_hbm, y_hbm, z_hbm, og, oc, xs, ys, zs, dmin, idxs_v, pub,
                red, cd, cix, dbuf, cd2, cix2, dbuf2, gxa, gya, gza, cxa,
                cya, cza, shared):
    c = lax.axis_index("c")
    s = lax.axis_index("s")
    b = c * 4 + s // 4
    q = s % 4
    qbase = q * _QUART
    lanes = lax.iota(jnp.int32, _L)

    pltpu.sync_copy(x_hbm.at[b], xs)
    pltpu.sync_copy(y_hbm.at[b], ys)
    pltpu.sync_copy(z_hbm.at[b], zs)

    def init_j(j, carry):
        dmin[pl.ds(j * _L, _L)] = jnp.full((_L,), 1e10, jnp.float32)
        return carry

    lax.fori_loop(0, _QUART // _L, init_j, 0)

    def step(i, carry):
        fi, acc = carry
        lane = lax.rem(i, _L)
        acc = jnp.where(lanes == lane, jnp.full((_L,), fi, jnp.int32), acc)

        @pl.when(lane == _L - 1)
        def _():
            idxs_v[pl.ds((i // _L) * _L, _L)] = acc

        fiv = jnp.full((_L,), fi, jnp.int32)
        cx = plsc.load_gather(xs, [fiv])
        cy = plsc.load_gather(ys, [fiv])
        cz = plsc.load_gather(zs, [fiv])

        _UPA = 4

        def upd(j, carry2):
            accs = list(carry2)
            for u in range(_UPA):
                lb = (j * _UPA + u) * _L
                px = xs[pl.ds(qbase + lb, _L)]
                py = ys[pl.ds(qbase + lb, _L)]
                pz = zs[pl.ds(qbase + lb, _L)]
                dx = px - cx
                dy = py - cy
                dz = pz - cz
                d = dx * dx + dy * dy
                d = d + dz * dz
                dm = jnp.minimum(dmin[pl.ds(lb, _L)], d)
                dmin[pl.ds(lb, _L)] = dm
                gi = lanes + (qbase + lb)
                rm, ri = accs[2 * u], accs[2 * u + 1]
                better = dm > rm
                accs[2 * u] = jnp.where(better, dm, rm)
                accs[2 * u + 1] = jnp.where(better, gi, ri)
            return tuple(accs)

        rmax0 = jnp.full((_L,), -jnp.inf, jnp.float32)
        ridx0 = jnp.zeros((_L,), jnp.int32)
        accs = lax.fori_loop(0, _QUART // _L // _UPA, upd,
                             (rmax0, ridx0) * _UPA)
        rmax, ridx = accs[0], accs[1]
        for u in range(1, _UPA):
            bv, bi2 = accs[2 * u], accs[2 * u + 1]
            take = (bv > rmax) | ((bv == rmax) & (bi2 < ridx))
            rmax = jnp.where(take, bv, rmax)
            ridx = jnp.where(take, bi2, ridx)

        m = jnp.max(rmax)
        cand = jnp.where(rmax == m, ridx, jnp.int32(2**30))
        mi = jnp.min(cand)

        pubv = jnp.where(lanes == 0, jnp.full((_L,), m, jnp.float32),
                         jnp.full((_L,), mi.astype(jnp.float32), jnp.float32))
        pub[...] = pubv
        par = lax.rem(i, 2) * (16 * _L)
        pltpu.sync_copy(pub, shared.at[pl.ds(par + s * _L, _L)])
        plsc.subcore_barrier()
        pltpu.sync_copy(shared.at[pl.ds(par + (s // 4) * 4 * _L, 4 * _L)],
                        red)

        def pick(j, carry2):
            bv, bi = carry2
            rv = red[pl.ds(j * _L, _L)]
            v = rv[0]
            ix = rv[1]
            better = v > bv
            bv = jnp.where(better, v, bv)
            bi = jnp.where(better, ix, bi)
            return bv, bi

        bv0 = jnp.float32(-jnp.inf)
        bi0 = jnp.float32(0.0)
        _, bi = lax.fori_loop(0, 4, pick, (bv0, bi0), unroll=4)
        return bi.astype(jnp.int32), acc

    lax.fori_loop(0, _G, step,
                  (jnp.int32(0), jnp.zeros((_L,), jnp.int32)))

    rowbase = q * _RPW

    def block(rb, carry0):
        civ = idxs_v[pl.ds(rowbase + rb * _L, _L)]
        bx = plsc.load_gather(xs, [civ])
        by = plsc.load_gather(ys, [civ])
        bz = plsc.load_gather(zs, [civ])
        cxa[pl.ds(rb * _L, _L)] = bx
        cya[pl.ds(rb * _L, _L)] = by
        cza[pl.ds(rb * _L, _L)] = bz

        def rowpair(jp, carry1):
            sela = lanes == 2 * jp
            selb = lanes == 2 * jp + 1
            cxva = jnp.full((_L,), jnp.sum(jnp.where(sela, bx, 0.0)))
            cyva = jnp.full((_L,), jnp.sum(jnp.where(sela, by, 0.0)))
            czva = jnp.full((_L,), jnp.sum(jnp.where(sela, bz, 0.0)))
            cxvb = jnp.full((_L,), jnp.sum(jnp.where(selb, bx, 0.0)))
            cyvb = jnp.full((_L,), jnp.sum(jnp.where(selb, by, 0.0)))
            czvb = jnp.full((_L,), jnp.sum(jnp.where(selb, bz, 0.0)))

            def fillA(v, carry2):
                cd[pl.ds(v * _L, _L)] = jnp.full((_L,), jnp.inf, jnp.float32)
                return carry2

            def fillB(v, carry2):
                cd2[pl.ds(v * _L, _L)] = jnp.full((_L,), jnp.inf,
                                                  jnp.float32)
                return carry2

            lax.fori_loop(0, _NV, fillA, 0)
            lax.fori_loop(0, _NV, fillB, 0)

            _UNR = 8
            inf16 = jnp.full((_L,), jnp.inf, jnp.float32)

            def p1(jo, carry2):
                accs = list(carry2)
                for ji in range(_UNR):
                    j = jo * _UNR + ji
                    px = xs[pl.ds(j * _L, _L)]
                    py = ys[pl.ds(j * _L, _L)]
                    pz = zs[pl.ds(j * _L, _L)]
                    dxa = cxva - px
                    dya = cyva - py
                    dza = czva - pz
                    da = dxa * dxa + dya * dya
                    da = da + dza * dza
                    dxb = cxvb - px
                    dyb = cyvb - py
                    dzb = czvb - pz
                    db = dxb * dxb + dyb * dyb
                    db = db + dzb * dzb
                    dbuf[pl.ds(j * _L, _L)] = da
                    dbuf2[pl.ds(j * _L, _L)] = db
                    o = 4 * (ji % 2)
                    m1a, m2a, m1b, m2b = accs[o:o + 4]
                    hia = jnp.maximum(m1a, da)
                    m1a = jnp.minimum(m1a, da)
                    m2a = jnp.minimum(m2a, hia)
                    hib = jnp.maximum(m1b, db)
                    m1b = jnp.minimum(m1b, db)
                    m2b = jnp.minimum(m2b, hib)
                    accs[o:o + 4] = [m1a, m2a, m1b, m2b]
                return tuple(accs)

            accs = lax.fori_loop(0, _N // _L // _UNR, p1, (inf16,) * 8)
            t0a = jnp.max(jnp.minimum(
                jnp.maximum(accs[0], accs[4]),
                jnp.minimum(accs[1], accs[5])))
            t0b = jnp.max(jnp.minimum(
                jnp.maximum(accs[2], accs[6]),
                jnp.minimum(accs[3], accs[7])))

            def p2(jo, carry2):
                cca, ta, ccb, tb = carry2

                def compactA(op):
                    R0k, R0i, R1k, R1i = _select32(cd, cix, lanes, _NV)
                    cd[pl.ds(0, _L)] = R0k
                    cix[pl.ds(0, _L)] = R0i
                    cd[pl.ds(_L, _L)] = R1k
                    cix[pl.ds(_L, _L)] = R1i
                    lax.fori_loop(2, _NV, fillA, 0)
                    return jnp.int32(2 * _L), R1k[15]

                def compactB(op):
                    R0k, R0i, R1k, R1i = _select32(cd2, cix2, lanes, _NV)
                    cd2[pl.ds(0, _L)] = R0k
                    cix2[pl.ds(0, _L)] = R0i
                    cd2[pl.ds(_L, _L)] = R1k
                    cix2[pl.ds(_L, _L)] = R1i
                    lax.fori_loop(2, _NV, fillB, 0)
                    return jnp.int32(2 * _L), R1k[15]

                cca, ta = lax.cond(cca > _CAP - _UNR * _L - _L, compactA,
                                   lambda op: (cca, ta), 0)
                ccb, tb = lax.cond(ccb > _CAP - _UNR * _L - _L, compactB,
                                   lambda op: (ccb, tb), 0)
                for ji in range(_UNR):
                    j = jo * _UNR + ji
                    gi = lanes + j * _L
                    da = dbuf[pl.ds(j * _L, _L)]
                    db = dbuf2[pl.ds(j * _L, _L)]
                    mska = da <= ta
                    mskb = db <= tb
                    na = plsc.all_reduce_population_count(mska)[0]
                    nb = plsc.all_reduce_population_count(mskb)[0]
                    plsc.store_compressed(cd.at[pl.ds(cca, _L)], da,
                                          mask=mska)
                    plsc.store_compressed(cix.at[pl.ds(cca, _L)], gi,
                                          mask=mska)
                    plsc.store_compressed(cd2.at[pl.ds(ccb, _L)], db,
                                          mask=mskb)
                    plsc.store_compressed(cix2.at[pl.ds(ccb, _L)], gi,
                                          mask=mskb)
                    cca = cca + na
                    ccb = ccb + nb
                return cca, ta, ccb, tb

            cca, _ta, ccb, _tb = lax.fori_loop(
                0, _N // _L // _UNR, p2,
                (jnp.int32(0), t0a, jnp.int32(0), t0b))

            for half, cdX, cixX, ccX, cxv, cyv, czv in (
                    (0, cd, cix, cca, cxva, cyva, czva),
                    (1, cd2, cix2, ccb, cxvb, cyvb, czvb)):
                R0k, R0i, R1k, R1i = _select32(cdX, cixX, lanes,
                                               (ccX + _L - 1) // _L)
                rl = rb * _L + 2 * jp + half
                gxa[pl.ds(rl * _M, _L)] = plsc.load_gather(xs, [R0i]) - cxv
                gya[pl.ds(rl * _M, _L)] = plsc.load_gather(ys, [R0i]) - cyv
                gza[pl.ds(rl * _M, _L)] = plsc.load_gather(zs, [R0i]) - czv
                gxa[pl.ds(rl * _M + _L, _L)] = (
                    plsc.load_gather(xs, [R1i]) - cxv)
                gya[pl.ds(rl * _M + _L, _L)] = (
                    plsc.load_gather(ys, [R1i]) - cyv)
                gza[pl.ds(rl * _M + _L, _L)] = (
                    plsc.load_gather(zs, [R1i]) - czv)
            return carry1

        lax.fori_loop(0, _L // 2, rowpair, 0)
        return carry0

    lax.fori_loop(0, _RPW // _L, block, 0)

    goff = (b * 3 * _G + rowbase) * _M
    pltpu.sync_copy(gxa, og.at[pl.ds(goff, _RPW * _M)])
    pltpu.sync_copy(gya, og.at[pl.ds(goff + _G * _M, _RPW * _M)])
    pltpu.sync_copy(gza, og.at[pl.ds(goff + 2 * _G * _M, _RPW * _M)])
    coff = b * 3 * _G + rowbase
    pltpu.sync_copy(cxa, oc.at[pl.ds(coff, _RPW)])
    pltpu.sync_copy(cya, oc.at[pl.ds(coff + _G, _RPW)])
    pltpu.sync_copy(cza, oc.at[pl.ds(coff + 2 * _G, _RPW)])


@jax.jit
def _group_sc(x, y, z):
    mesh = plsc.VectorSubcoreMesh(core_axis_name="c", subcore_axis_name="s")
    return pl.kernel(
        _group_body,
        out_type=(
            jax.ShapeDtypeStruct((_B * 3 * _G * _M,), jnp.float32),
            jax.ShapeDtypeStruct((_B * 3 * _G,), jnp.float32),
        ),
        mesh=mesh,
        compiler_params=pltpu.CompilerParams(needs_layout_passes=False),
        scratch_types=[
            pltpu.VMEM((_N,), jnp.float32),
            pltpu.VMEM((_N,), jnp.float32),
            pltpu.VMEM((_N,), jnp.float32),
            pltpu.VMEM((_QUART,), jnp.float32),
            pltpu.VMEM((_G,), jnp.int32),
            pltpu.VMEM((_L,), jnp.float32),
            pltpu.VMEM((4 * _L,), jnp.float32),
            pltpu.VMEM((_CAP,), jnp.float32),
            pltpu.VMEM((_CAP,), jnp.int32),
            pltpu.VMEM((_N,), jnp.float32),
            pltpu.VMEM((_CAP,), jnp.float32),
            pltpu.VMEM((_CAP,), jnp.int32),
            pltpu.VMEM((_N,), jnp.float32),
            pltpu.VMEM((_RPW * _M,), jnp.float32),
            pltpu.VMEM((_RPW * _M,), jnp.float32),
            pltpu.VMEM((_RPW * _M,), jnp.float32),
            pltpu.VMEM((_RPW,), jnp.float32),
            pltpu.VMEM((_RPW,), jnp.float32),
            pltpu.VMEM((_RPW,), jnp.float32),
            pltpu.VMEM_SHARED((32 * _L,), jnp.float32),
        ],
    )(x, y, z)


def kernel(xyz):
    xyzT = jnp.transpose(xyz, (0, 2, 1))
    x = xyzT[:, 0]
    y = xyzT[:, 1]
    z = xyzT[:, 2]
    og, oc = _group_sc(x, y, z)
    groups = jnp.transpose(og.reshape(_B, 3, _G, _M), (0, 2, 3, 1))
    center = jnp.transpose(oc.reshape(_B, 3, _G), (0, 2, 1))
    return groups, center


# speedup vs baseline: 1.0142x; 1.0142x over previous
import jax
import jax.numpy as jnp
from jax import lax
from jax.experimental import pallas as pl
from jax.experimental.pallas import tpu as pltpu
from jax.experimental.pallas import tpu_sc as plsc

_B, _N, _G, _M = 8, 8192, 512, 32
_L = 16
_QUART = _N // 4
_RPW = _G // 4
_CAP = 512
_NV = _CAP // _L


def _permute(v, idx):
    return lax.gather(
        v, idx[:, None],
        lax.GatherDimensionNumbers(offset_dims=(), collapsed_slice_dims=(0,),
                                   start_index_map=(0,)),
        (1,), mode=lax.GatherScatterMode.PROMISE_IN_BOUNDS)


def _cmpex(vk, vi, stride, lanes, dirm=None):
    pidx = lanes ^ stride
    pk = _permute(vk, pidx)
    pi = _permute(vi, pidx)
    less = (pk < vk) | ((pk == vk) & (pi < vi))
    is_high = (lanes & stride) != 0
    take_self = less == is_high
    if dirm is not None:
        take_self = take_self != dirm
    return jnp.where(take_self, vk, pk), jnp.where(take_self, vi, pi)


def _sort16(vk, vi, lanes):
    for k in (2, 4, 8, 16):
        dirm = (lanes & k) != 0
        stride = k // 2
        while stride >= 1:
            vk, vi = _cmpex(vk, vi, stride, lanes, dirm)
            stride //= 2
    return vk, vi


def _minmaxpair(ak, ai, bk, bi):
    bless = (bk < ak) | ((bk == ak) & (bi < ai))
    return (jnp.where(bless, bk, ak), jnp.where(bless, bi, ai),
            jnp.where(bless, ak, bk), jnp.where(bless, ai, bi))


def _merge(R, sk, si, lanes):
    R0k, R0i, R1k, R1i = R
    rev = 15 - lanes
    rk = _permute(sk, rev)
    ri = _permute(si, rev)
    l1k, l1i, _, _ = _minmaxpair(R1k, R1i, rk, ri)
    ak, ai, bk, bi = _minmaxpair(R0k, R0i, l1k, l1i)
    for stride in (8, 4, 2, 1):
        ak, ai = _cmpex(ak, ai, stride, lanes)
        bk, bi = _cmpex(bk, bi, stride, lanes)
    return ak, ai, bk, bi


def _select32(cd, cix, lanes, nv):
    def mbody(v, R):
        kd = cd[pl.ds(v * _L, _L)]
        ki = cix[pl.ds(v * _L, _L)]
        worst = R[2][15]

        def do(Rin):
            sk, si = _sort16(kd, ki, lanes)
            return _merge(Rin, sk, si, lanes)

        return lax.cond(jnp.min(kd) <= worst, do, lambda Rin: Rin, R)

    inf = jnp.full((_L,), jnp.inf, jnp.float32)
    zero = jnp.zeros((_L,), jnp.int32)
    return lax.fori_loop(0, nv, mbody, (inf, zero, inf, zero))


def _group_body(x_hbm, y_hbm, z_hbm, og, oc, xs, ys, zs, dmin, idxs_v, pub,
                red, cd, cix, dbuf, cd2, cix2, dbuf2, gxa, gya, gza, cxa,
                cya, cza, shared):
    c = lax.axis_index("c")
    s = lax.axis_index("s")
    b = c * 4 + s // 4
    q = s % 4
    qbase = q * _QUART
    lanes = lax.iota(jnp.int32, _L)

    pltpu.sync_copy(x_hbm.at[b], xs)
    pltpu.sync_copy(y_hbm.at[b], ys)
    pltpu.sync_copy(z_hbm.at[b], zs)

    def init_j(j, carry):
        dmin[pl.ds(j * _L, _L)] = jnp.full((_L,), 1e10, jnp.float32)
        return carry

    lax.fori_loop(0, _QUART // _L, init_j, 0)

    def step(i, carry):
        fi, acc = carry
        lane = lax.rem(i, _L)
        acc = jnp.where(lanes == lane, jnp.full((_L,), fi, jnp.int32), acc)

        @pl.when(lane == _L - 1)
        def _():
            idxs_v[pl.ds((i // _L) * _L, _L)] = acc

        fiv = jnp.full((_L,), fi, jnp.int32)
        cx = plsc.load_gather(xs, [fiv])
        cy = plsc.load_gather(ys, [fiv])
        cz = plsc.load_gather(zs, [fiv])

        _UPA = 4

        def upd(j, carry2):
            accs = list(carry2)
            for u in range(_UPA):
                lb = (j * _UPA + u) * _L
                px = xs[pl.ds(qbase + lb, _L)]
                py = ys[pl.ds(qbase + lb, _L)]
                pz = zs[pl.ds(qbase + lb, _L)]
                dx = px - cx
                dy = py - cy
                dz = pz - cz
                d = dx * dx + dy * dy
                d = d + dz * dz
                dm = jnp.minimum(dmin[pl.ds(lb, _L)], d)
                dmin[pl.ds(lb, _L)] = dm
                gi = lanes + (qbase + lb)
                rm, ri = accs[2 * u], accs[2 * u + 1]
                better = dm > rm
                accs[2 * u] = jnp.where(better, dm, rm)
                accs[2 * u + 1] = jnp.where(better, gi, ri)
            return tuple(accs)

        rmax0 = jnp.full((_L,), -jnp.inf, jnp.float32)
        ridx0 = jnp.zeros((_L,), jnp.int32)
        accs = lax.fori_loop(0, _QUART // _L // _UPA, upd,
                             (rmax0, ridx0) * _UPA)
        rmax, ridx = accs[0], accs[1]
        for u in range(1, _UPA):
            bv, bi2 = accs[2 * u], accs[2 * u + 1]
            take = (bv > rmax) | ((bv == rmax) & (bi2 < ridx))
            rmax = jnp.where(take, bv, rmax)
            ridx = jnp.where(take, bi2, ridx)

        m = jnp.max(rmax)
        cand = jnp.where(rmax == m, ridx, jnp.int32(2**30))
        mi = jnp.min(cand)

        pubv = jnp.where(lanes == 0, jnp.full((_L,), m, jnp.float32),
                         jnp.full((_L,), mi.astype(jnp.float32), jnp.float32))
        pub[...] = pubv
        par = lax.rem(i, 2) * (16 * _L)
        pltpu.sync_copy(pub, shared.at[pl.ds(par + s * _L, _L)])
        plsc.subcore_barrier()
        pltpu.sync_copy(shared.at[pl.ds(par + (s // 4) * 4 * _L, 4 * _L)],
                        red)

        def pick(j, carry2):
            bv, bi = carry2
            rv = red[pl.ds(j * _L, _L)]
            v = rv[0]
            ix = rv[1]
            better = v > bv
            bv = jnp.where(better, v, bv)
            bi = jnp.where(better, ix, bi)
            return bv, bi

        bv0 = jnp.float32(-jnp.inf)
        bi0 = jnp.float32(0.0)
        _, bi = lax.fori_loop(0, 4, pick, (bv0, bi0), unroll=4)
        return bi.astype(jnp.int32), acc

    lax.fori_loop(0, _G, step,
                  (jnp.int32(0), jnp.zeros((_L,), jnp.int32)))

    rowbase = q * _RPW

    def block(rb, carry0):
        civ = idxs_v[pl.ds(rowbase + rb * _L, _L)]
        bx = plsc.load_gather(xs, [civ])
        by = plsc.load_gather(ys, [civ])
        bz = plsc.load_gather(zs, [civ])
        cxa[pl.ds(rb * _L, _L)] = bx
        cya[pl.ds(rb * _L, _L)] = by
        cza[pl.ds(rb * _L, _L)] = bz

        def rowpair(jp, carry1):
            sela = lanes == 2 * jp
            selb = lanes == 2 * jp + 1
            cxva = jnp.full((_L,), jnp.sum(jnp.where(sela, bx, 0.0)))
            cyva = jnp.full((_L,), jnp.sum(jnp.where(sela, by, 0.0)))
            czva = jnp.full((_L,), jnp.sum(jnp.where(sela, bz, 0.0)))
            cxvb = jnp.full((_L,), jnp.sum(jnp.where(selb, bx, 0.0)))
            cyvb = jnp.full((_L,), jnp.sum(jnp.where(selb, by, 0.0)))
            czvb = jnp.full((_L,), jnp.sum(jnp.where(selb, bz, 0.0)))

            def fillA(v, carry2):
                cd[pl.ds(v * _L, _L)] = jnp.full((_L,), jnp.inf, jnp.float32)
                return carry2

            def fillB(v, carry2):
                cd2[pl.ds(v * _L, _L)] = jnp.full((_L,), jnp.inf,
                                                  jnp.float32)
                return carry2

            lax.fori_loop(0, _NV, fillA, 0)
            lax.fori_loop(0, _NV, fillB, 0)

            _UNR = 8
            inf16 = jnp.full((_L,), jnp.inf, jnp.float32)

            def p1(jo, carry2):
                accs = list(carry2)
                for ji in range(_UNR):
                    j = jo * _UNR + ji
                    px = xs[pl.ds(j * _L, _L)]
                    py = ys[pl.ds(j * _L, _L)]
                    pz = zs[pl.ds(j * _L, _L)]
                    dxa = cxva - px
                    dya = cyva - py
                    dza = czva - pz
                    da = dxa * dxa + dya * dya
                    da = da + dza * dza
                    dxb = cxvb - px
                    dyb = cyvb - py
                    dzb = czvb - pz
                    db = dxb * dxb + dyb * dyb
                    db = db + dzb * dzb
                    dbuf[pl.ds(j * _L, _L)] = da
                    dbuf2[pl.ds(j * _L, _L)] = db
                    m1a, m2a, m1b, m2b = accs
                    hia = jnp.maximum(m1a, da)
                    m1a = jnp.minimum(m1a, da)
                    m2a = jnp.minimum(m2a, hia)
                    hib = jnp.maximum(m1b, db)
                    m1b = jnp.minimum(m1b, db)
                    m2b = jnp.minimum(m2b, hib)
                    accs = [m1a, m2a, m1b, m2b]
                return tuple(accs)

            _, m2a, _, m2b = lax.fori_loop(0, _N // _L // _UNR, p1,
                                           (inf16,) * 4)
            t0a = jnp.max(m2a)
            t0b = jnp.max(m2b)

            def p2(jo, carry2):
                cca, ta, ccb, tb = carry2

                def compactA(op):
                    R0k, R0i, R1k, R1i = _select32(cd, cix, lanes, _NV)
                    cd[pl.ds(0, _L)] = R0k
                    cix[pl.ds(0, _L)] = R0i
                    cd[pl.ds(_L, _L)] = R1k
                    cix[pl.ds(_L, _L)] = R1i
                    lax.fori_loop(2, _NV, fillA, 0)
                    return jnp.int32(2 * _L), R1k[15]

                def compactB(op):
                    R0k, R0i, R1k, R1i = _select32(cd2, cix2, lanes, _NV)
                    cd2[pl.ds(0, _L)] = R0k
                    cix2[pl.ds(0, _L)] = R0i
                    cd2[pl.ds(_L, _L)] = R1k
                    cix2[pl.ds(_L, _L)] = R1i
                    lax.fori_loop(2, _NV, fillB, 0)
                    return jnp.int32(2 * _L), R1k[15]

                cca, ta = lax.cond(cca > _CAP - _UNR * _L - _L, compactA,
                                   lambda op: (cca, ta), 0)
                ccb, tb = lax.cond(ccb > _CAP - _UNR * _L - _L, compactB,
                                   lambda op: (ccb, tb), 0)
                for ji in range(_UNR):
                    j = jo * _UNR + ji
                    gi = lanes + j * _L
                    da = dbuf[pl.ds(j * _L, _L)]
                    db = dbuf2[pl.ds(j * _L, _L)]
                    mska = da <= ta
                    mskb = db <= tb
                    na = plsc.all_reduce_population_count(mska)[0]
                    nb = plsc.all_reduce_population_count(mskb)[0]
                    plsc.store_compressed(cd.at[pl.ds(cca, _L)], da,
                                          mask=mska)
                    plsc.store_compressed(cix.at[pl.ds(cca, _L)], gi,
                                          mask=mska)
                    plsc.store_compressed(cd2.at[pl.ds(ccb, _L)], db,
                                          mask=mskb)
                    plsc.store_compressed(cix2.at[pl.ds(ccb, _L)], gi,
                                          mask=mskb)
                    cca = cca + na
                    ccb = ccb + nb
                return cca, ta, ccb, tb

            cca, _ta, ccb, _tb = lax.fori_loop(
                0, _N // _L // _UNR, p2,
                (jnp.int32(0), t0a, jnp.int32(0), t0b))

            for half, cdX, cixX, ccX, cxv, cyv, czv in (
                    (0, cd, cix, cca, cxva, cyva, czva),
                    (1, cd2, cix2, ccb, cxvb, cyvb, czvb)):
                R0k, R0i, R1k, R1i = _select32(cdX, cixX, lanes,
                                               (ccX + _L - 1) // _L)
                rl = rb * _L + 2 * jp + half
                gxa[pl.ds(rl * _M, _L)] = plsc.load_gather(xs, [R0i]) - cxv
                gya[pl.ds(rl * _M, _L)] = plsc.load_gather(ys, [R0i]) - cyv
                gza[pl.ds(rl * _M, _L)] = plsc.load_gather(zs, [R0i]) - czv
                gxa[pl.ds(rl * _M + _L, _L)] = (
                    plsc.load_gather(xs, [R1i]) - cxv)
                gya[pl.ds(rl * _M + _L, _L)] = (
                    plsc.load_gather(ys, [R1i]) - cyv)
                gza[pl.ds(rl * _M + _L, _L)] = (
                    plsc.load_gather(zs, [R1i]) - czv)
            return carry1

        lax.fori_loop(0, _L // 2, rowpair, 0)
        return carry0

    lax.fori_loop(0, _RPW // _L, block, 0)

    goff = (b * 3 * _G + rowbase) * _M
    pltpu.sync_copy(gxa, og.at[pl.ds(goff, _RPW * _M)])
    pltpu.sync_copy(gya, og.at[pl.ds(goff + _G * _M, _RPW * _M)])
    pltpu.sync_copy(gza, og.at[pl.ds(goff + 2 * _G * _M, _RPW * _M)])
    coff = b * 3 * _G + rowbase
    pltpu.sync_copy(cxa, oc.at[pl.ds(coff, _RPW)])
    pltpu.sync_copy(cya, oc.at[pl.ds(coff + _G, _RPW)])
    pltpu.sync_copy(cza, oc.at[pl.ds(coff + 2 * _G, _RPW)])


@jax.jit
def _group_sc(x, y, z):
    mesh = plsc.VectorSubcoreMesh(core_axis_name="c", subcore_axis_name="s")
    return pl.kernel(
        _group_body,
        out_type=(
            jax.ShapeDtypeStruct((_B * 3 * _G * _M,), jnp.float32),
            jax.ShapeDtypeStruct((_B * 3 * _G,), jnp.float32),
        ),
        mesh=mesh,
        compiler_params=pltpu.CompilerParams(needs_layout_passes=False),
        scratch_types=[
            pltpu.VMEM((_N,), jnp.float32),
            pltpu.VMEM((_N,), jnp.float32),
            pltpu.VMEM((_N,), jnp.float32),
            pltpu.VMEM((_QUART,), jnp.float32),
            pltpu.VMEM((_G,), jnp.int32),
            pltpu.VMEM((_L,), jnp.float32),
            pltpu.VMEM((4 * _L,), jnp.float32),
            pltpu.VMEM((_CAP,), jnp.float32),
            pltpu.VMEM((_CAP,), jnp.int32),
            pltpu.VMEM((_N,), jnp.float32),
            pltpu.VMEM((_CAP,), jnp.float32),
            pltpu.VMEM((_CAP,), jnp.int32),
            pltpu.VMEM((_N,), jnp.float32),
            pltpu.VMEM((_RPW * _M,), jnp.float32),
            pltpu.VMEM((_RPW * _M,), jnp.float32),
            pltpu.VMEM((_RPW * _M,), jnp.float32),
            pltpu.VMEM((_RPW,), jnp.float32),
            pltpu.VMEM((_RPW,), jnp.float32),
            pltpu.VMEM((_RPW,), jnp.float32),
            pltpu.VMEM_SHARED((32 * _L,), jnp.float32),
        ],
    )(x, y, z)


def kernel(xyz):
    xyzT = jnp.transpose(xyz, (0, 2, 1))
    x = xyzT[:, 0]
    y = xyzT[:, 1]
    z = xyzT[:, 2]
    og, oc = _group_sc(x, y, z)
    groups = jnp.transpose(og.reshape(_B, 3, _G, _M), (0, 2, 3, 1))
    center = jnp.transpose(oc.reshape(_B, 3, _G), (0, 2, 1))
    return groups, center


# speedup vs baseline: 1.0486x; 1.0340x over previous
import jax
import jax.numpy as jnp
from jax import lax
from jax.experimental import pallas as pl
from jax.experimental.pallas import tpu as pltpu
from jax.experimental.pallas import tpu_sc as plsc

_B, _N, _G, _M = 8, 8192, 512, 32
_L = 16
_QUART = _N // 4
_RPW = _G // 4
_CAP = 512
_NV = _CAP // _L


def _permute(v, idx):
    return lax.gather(
        v, idx[:, None],
        lax.GatherDimensionNumbers(offset_dims=(), collapsed_slice_dims=(0,),
                                   start_index_map=(0,)),
        (1,), mode=lax.GatherScatterMode.PROMISE_IN_BOUNDS)


def _cmpex(vk, vi, stride, lanes, dirm=None):
    pidx = lanes ^ stride
    pk = _permute(vk, pidx)
    pi = _permute(vi, pidx)
    less = (pk < vk) | ((pk == vk) & (pi < vi))
    is_high = (lanes & stride) != 0
    take_self = less == is_high
    if dirm is not None:
        take_self = take_self != dirm
    return jnp.where(take_self, vk, pk), jnp.where(take_self, vi, pi)


def _sort16(vk, vi, lanes):
    for k in (2, 4, 8, 16):
        dirm = (lanes & k) != 0
        stride = k // 2
        while stride >= 1:
            vk, vi = _cmpex(vk, vi, stride, lanes, dirm)
            stride //= 2
    return vk, vi


def _minmaxpair(ak, ai, bk, bi):
    bless = (bk < ak) | ((bk == ak) & (bi < ai))
    return (jnp.where(bless, bk, ak), jnp.where(bless, bi, ai),
            jnp.where(bless, ak, bk), jnp.where(bless, ai, bi))


def _merge(R, sk, si, lanes):
    R0k, R0i, R1k, R1i = R
    rev = 15 - lanes
    rk = _permute(sk, rev)
    ri = _permute(si, rev)
    l1k, l1i, _, _ = _minmaxpair(R1k, R1i, rk, ri)
    ak, ai, bk, bi = _minmaxpair(R0k, R0i, l1k, l1i)
    for stride in (8, 4, 2, 1):
        ak, ai = _cmpex(ak, ai, stride, lanes)
        bk, bi = _cmpex(bk, bi, stride, lanes)
    return ak, ai, bk, bi


def _select32(cd, cix, lanes, nv):
    def mbody(v, R):
        kd = cd[pl.ds(v * _L, _L)]
        ki = cix[pl.ds(v * _L, _L)]
        worst = R[2][15]

        def do(Rin):
            sk, si = _sort16(kd, ki, lanes)
            return _merge(Rin, sk, si, lanes)

        return lax.cond(jnp.min(kd) <= worst, do, lambda Rin: Rin, R)

    inf = jnp.full((_L,), jnp.inf, jnp.float32)
    zero = jnp.zeros((_L,), jnp.int32)
    return lax.fori_loop(0, nv, mbody, (inf, zero, inf, zero))


def _group_body(x_hbm, y_hbm, z_hbm, og, oc, xs, ys, zs, dmin, idxs_v, pub,
                red, cd, cix, dbuf, cd2, cix2, dbuf2, gxa, gya, gza, cxa,
                cya, cza, shared):
    c = lax.axis_index("c")
    s = lax.axis_index("s")
    b = c * 4 + s // 4
    q = s % 4
    qbase = q * _QUART
    lanes = lax.iota(jnp.int32, _L)

    pltpu.sync_copy(x_hbm.at[b], xs)
    pltpu.sync_copy(y_hbm.at[b], ys)
    pltpu.sync_copy(z_hbm.at[b], zs)

    def init_j(j, carry):
        dmin[pl.ds(j * _L, _L)] = jnp.full((_L,), 1e10, jnp.float32)
        return carry

    lax.fori_loop(0, _QUART // _L, init_j, 0)

    def step(i, carry):
        fi, acc = carry
        lane = lax.rem(i, _L)
        acc = jnp.where(lanes == lane, jnp.full((_L,), fi, jnp.int32), acc)

        @pl.when(lane == _L - 1)
        def _():
            idxs_v[pl.ds((i // _L) * _L, _L)] = acc

        fiv = jnp.full((_L,), fi, jnp.int32)
        cx = plsc.load_gather(xs, [fiv])
        cy = plsc.load_gather(ys, [fiv])
        cz = plsc.load_gather(zs, [fiv])

        def upd(j, carry2):
            rmax, ridx = carry2
            lb = j * _L
            px = xs[pl.ds(qbase + lb, _L)]
            py = ys[pl.ds(qbase + lb, _L)]
            pz = zs[pl.ds(qbase + lb, _L)]
            dx = px - cx
            dy = py - cy
            dz = pz - cz
            d = dx * dx + dy * dy
            d = d + dz * dz
            dm = jnp.minimum(dmin[pl.ds(lb, _L)], d)
            dmin[pl.ds(lb, _L)] = dm
            gi = lanes + (qbase + lb)
            better = dm > rmax
            rmax = jnp.where(better, dm, rmax)
            ridx = jnp.where(better, gi, ridx)
            return rmax, ridx

        rmax0 = jnp.full((_L,), -jnp.inf, jnp.float32)
        ridx0 = jnp.zeros((_L,), jnp.int32)
        rmax, ridx = lax.fori_loop(0, _QUART // _L, upd, (rmax0, ridx0),
                                   unroll=8)

        m = jnp.max(rmax)
        cand = jnp.where(rmax == m, ridx, jnp.int32(2**30))
        mi = jnp.min(cand)

        pubv = jnp.where(lanes == 0, jnp.full((_L,), m, jnp.float32),
                         jnp.full((_L,), mi.astype(jnp.float32), jnp.float32))
        pub[...] = pubv
        par = lax.rem(i, 2) * (16 * _L)
        pltpu.sync_copy(pub, shared.at[pl.ds(par + s * _L, _L)])
        plsc.subcore_barrier()
        pltpu.sync_copy(shared.at[pl.ds(par + (s // 4) * 4 * _L, 4 * _L)],
                        red)

        def pick(j, carry2):
            bv, bi = carry2
            rv = red[pl.ds(j * _L, _L)]
            v = rv[0]
            ix = rv[1]
            better = v > bv
            bv = jnp.where(better, v, bv)
            bi = jnp.where(better, ix, bi)
            return bv, bi

        bv0 = jnp.float32(-jnp.inf)
        bi0 = jnp.float32(0.0)
        _, bi = lax.fori_loop(0, 4, pick, (bv0, bi0), unroll=4)
        return bi.astype(jnp.int32), acc

    lax.fori_loop(0, _G, step,
                  (jnp.int32(0), jnp.zeros((_L,), jnp.int32)))

    rowbase = q * _RPW

    def block(rb, carry0):
        civ = idxs_v[pl.ds(rowbase + rb * _L, _L)]
        bx = plsc.load_gather(xs, [civ])
        by = plsc.load_gather(ys, [civ])
        bz = plsc.load_gather(zs, [civ])
        cxa[pl.ds(rb * _L, _L)] = bx
        cya[pl.ds(rb * _L, _L)] = by
        cza[pl.ds(rb * _L, _L)] = bz

        def rowpair(jp, carry1):
            sela = lanes == 2 * jp
            selb = lanes == 2 * jp + 1
            cxva = jnp.full((_L,), jnp.sum(jnp.where(sela, bx, 0.0)))
            cyva = jnp.full((_L,), jnp.sum(jnp.where(sela, by, 0.0)))
            czva = jnp.full((_L,), jnp.sum(jnp.where(sela, bz, 0.0)))
            cxvb = jnp.full((_L,), jnp.sum(jnp.where(selb, bx, 0.0)))
            cyvb = jnp.full((_L,), jnp.sum(jnp.where(selb, by, 0.0)))
            czvb = jnp.full((_L,), jnp.sum(jnp.where(selb, bz, 0.0)))

            def fillA(v, carry2):
                cd[pl.ds(v * _L, _L)] = jnp.full((_L,), jnp.inf, jnp.float32)
                return carry2

            def fillB(v, carry2):
                cd2[pl.ds(v * _L, _L)] = jnp.full((_L,), jnp.inf,
                                                  jnp.float32)
                return carry2

            lax.fori_loop(0, _NV, fillA, 0)
            lax.fori_loop(0, _NV, fillB, 0)

            _UNR = 8
            inf16 = jnp.full((_L,), jnp.inf, jnp.float32)

            def p1(jo, carry2):
                accs = list(carry2)
                for ji in range(_UNR):
                    j = jo * _UNR + ji
                    px = xs[pl.ds(j * _L, _L)]
                    py = ys[pl.ds(j * _L, _L)]
                    pz = zs[pl.ds(j * _L, _L)]
                    dxa = cxva - px
                    dya = cyva - py
                    dza = czva - pz
                    da = dxa * dxa + dya * dya
                    da = da + dza * dza
                    dxb = cxvb - px
                    dyb = cyvb - py
                    dzb = czvb - pz
                    db = dxb * dxb + dyb * dyb
                    db = db + dzb * dzb
                    dbuf[pl.ds(j * _L, _L)] = da
                    dbuf2[pl.ds(j * _L, _L)] = db
                    m1a, m2a, m1b, m2b = accs
                    hia = jnp.maximum(m1a, da)
                    m1a = jnp.minimum(m1a, da)
                    m2a = jnp.minimum(m2a, hia)
                    hib = jnp.maximum(m1b, db)
                    m1b = jnp.minimum(m1b, db)
                    m2b = jnp.minimum(m2b, hib)
                    accs = [m1a, m2a, m1b, m2b]
                return tuple(accs)

            _, m2a, _, m2b = lax.fori_loop(0, _N // _L // _UNR, p1,
                                           (inf16,) * 4)
            t0a = jnp.max(m2a)
            t0b = jnp.max(m2b)

            def p2(jo, carry2):
                cca, ta, ccb, tb = carry2

                def compactA(op):
                    R0k, R0i, R1k, R1i = _select32(cd, cix, lanes, _NV)
                    cd[pl.ds(0, _L)] = R0k
                    cix[pl.ds(0, _L)] = R0i
                    cd[pl.ds(_L, _L)] = R1k
                    cix[pl.ds(_L, _L)] = R1i
                    lax.fori_loop(2, _NV, fillA, 0)
                    return jnp.int32(2 * _L), R1k[15]

                def compactB(op):
                    R0k, R0i, R1k, R1i = _select32(cd2, cix2, lanes, _NV)
                    cd2[pl.ds(0, _L)] = R0k
                    cix2[pl.ds(0, _L)] = R0i
                    cd2[pl.ds(_L, _L)] = R1k
                    cix2[pl.ds(_L, _L)] = R1i
                    lax.fori_loop(2, _NV, fillB, 0)
                    return jnp.int32(2 * _L), R1k[15]

                cca, ta = lax.cond(cca > _CAP - _UNR * _L - _L, compactA,
                                   lambda op: (cca, ta), 0)
                ccb, tb = lax.cond(ccb > _CAP - _UNR * _L - _L, compactB,
                                   lambda op: (ccb, tb), 0)
                for ji in range(_UNR):
                    j = jo * _UNR + ji
                    gi = lanes + j * _L
                    da = dbuf[pl.ds(j * _L, _L)]
                    db = dbuf2[pl.ds(j * _L, _L)]
                    mska = da <= ta
                    mskb = db <= tb
                    na = plsc.all_reduce_population_count(mska)[0]
                    nb = plsc.all_reduce_population_count(mskb)[0]
                    plsc.store_compressed(cd.at[pl.ds(cca, _L)], da,
                                          mask=mska)
                    plsc.store_compressed(cix.at[pl.ds(cca, _L)], gi,
                                          mask=mska)
                    plsc.store_compressed(cd2.at[pl.ds(ccb, _L)], db,
                                          mask=mskb)
                    plsc.store_compressed(cix2.at[pl.ds(ccb, _L)], gi,
                                          mask=mskb)
                    cca = cca + na
                    ccb = ccb + nb
                return cca, ta, ccb, tb

            cca, _ta, ccb, _tb = lax.fori_loop(
                0, _N // _L // _UNR, p2,
                (jnp.int32(0), t0a, jnp.int32(0), t0b))

            for half, cdX, cixX, ccX, cxv, cyv, czv in (
                    (0, cd, cix, cca, cxva, cyva, czva),
                    (1, cd2, cix2, ccb, cxvb, cyvb, czvb)):
                R0k, R0i, R1k, R1i = _select32(cdX, cixX, lanes,
                                               (ccX + _L - 1) // _L)
                rl = rb * _L + 2 * jp + half
                gxa[pl.ds(rl * _M, _L)] = plsc.load_gather(xs, [R0i]) - cxv
                gya[pl.ds(rl * _M, _L)] = plsc.load_gather(ys, [R0i]) - cyv
                gza[pl.ds(rl * _M, _L)] = plsc.load_gather(zs, [R0i]) - czv
                gxa[pl.ds(rl * _M + _L, _L)] = (
                    plsc.load_gather(xs, [R1i]) - cxv)
                gya[pl.ds(rl * _M + _L, _L)] = (
                    plsc.load_gather(ys, [R1i]) - cyv)
                gza[pl.ds(rl * _M + _L, _L)] = (
                    plsc.load_gather(zs, [R1i]) - czv)
            return carry1

        lax.fori_loop(0, _L // 2, rowpair, 0)
        return carry0

    lax.fori_loop(0, _RPW // _L, block, 0)

    goff = (b * 3 * _G + rowbase) * _M
    pltpu.sync_copy(gxa, og.at[pl.ds(goff, _RPW * _M)])
    pltpu.sync_copy(gya, og.at[pl.ds(goff + _G * _M, _RPW * _M)])
    pltpu.sync_copy(gza, og.at[pl.ds(goff + 2 * _G * _M, _RPW * _M)])
    coff = b * 3 * _G + rowbase
    pltpu.sync_copy(cxa, oc.at[pl.ds(coff, _RPW)])
    pltpu.sync_copy(cya, oc.at[pl.ds(coff + _G, _RPW)])
    pltpu.sync_copy(cza, oc.at[pl.ds(coff + 2 * _G, _RPW)])


@jax.jit
def _group_sc(x, y, z):
    mesh = plsc.VectorSubcoreMesh(core_axis_name="c", subcore_axis_name="s")
    return pl.kernel(
        _group_body,
        out_type=(
            jax.ShapeDtypeStruct((_B * 3 * _G * _M,), jnp.float32),
            jax.ShapeDtypeStruct((_B * 3 * _G,), jnp.float32),
        ),
        mesh=mesh,
        compiler_params=pltpu.CompilerParams(needs_layout_passes=False),
        scratch_types=[
            pltpu.VMEM((_N,), jnp.float32),
            pltpu.VMEM((_N,), jnp.float32),
            pltpu.VMEM((_N,), jnp.float32),
            pltpu.VMEM((_QUART,), jnp.float32),
            pltpu.VMEM((_G,), jnp.int32),
            pltpu.VMEM((_L,), jnp.float32),
            pltpu.VMEM((4 * _L,), jnp.float32),
            pltpu.VMEM((_CAP,), jnp.float32),
            pltpu.VMEM((_CAP,), jnp.int32),
            pltpu.VMEM((_N,), jnp.float32),
            pltpu.VMEM((_CAP,), jnp.float32),
            pltpu.VMEM((_CAP,), jnp.int32),
            pltpu.VMEM((_N,), jnp.float32),
            pltpu.VMEM((_RPW * _M,), jnp.float32),
            pltpu.VMEM((_RPW * _M,), jnp.float32),
            pltpu.VMEM((_RPW * _M,), jnp.float32),
            pltpu.VMEM((_RPW,), jnp.float32),
            pltpu.VMEM((_RPW,), jnp.float32),
            pltpu.VMEM((_RPW,), jnp.float32),
            pltpu.VMEM_SHARED((32 * _L,), jnp.float32),
        ],
    )(x, y, z)


def kernel(xyz):
    xyzT = jnp.transpose(xyz, (0, 2, 1))
    x = xyzT[:, 0]
    y = xyzT[:, 1]
    z = xyzT[:, 2]
    og, oc = _group_sc(x, y, z)
    groups = jnp.transpose(og.reshape(_B, 3, _G, _M), (0, 2, 3, 1))
    center = jnp.transpose(oc.reshape(_B, 3, _G), (0, 2, 1))
    return groups, center
